# baseline (speedup 1.0000x reference)
import functools

import jax
import jax.numpy as jnp
import numpy as np
from jax import lax
from jax.experimental import pallas as pl
from jax.experimental.pallas import tpu as pltpu
from jax.experimental.pallas import tpu_sc as plsc

N = 50000
E = 800000
HID = 64
L = 4
AVG_D_LOG = float(np.log(16.0))

W = 256
NB = 196
NP = NB * W
NW = 32
EC = E // NW
EP2 = E + NB * 8
EP3 = EP2 + 512
BIGF = 3.0e38

MESH = plsc.VectorSubcoreMesh(core_axis_name="c", subcore_axis_name="s")
SC_PARAMS = pltpu.CompilerParams(use_tc_tiling_on_sc=False,
                                 needs_layout_passes=False)

_IOTA16 = tuple(range(16))


def _al(x):
    return pl.multiple_of(x, 8)


def _wid():
    return lax.axis_index("s") * 2 + lax.axis_index("c")


@functools.partial(
    pl.kernel,
    out_type=(
        jax.ShapeDtypeStruct((NP, HID), jnp.float32),
        jax.ShapeDtypeStruct((NW, NP + 16), jnp.int32),
    ),
    mesh=MESH,
    compiler_params=SC_PARAMS,
    scratch_types=[
        pltpu.VMEM((392,), jnp.int32),
        pltpu.VMEM((392, HID), jnp.float32),
        pltpu.VMEM((NP + 16,), jnp.int32),
        pltpu.VMEM((2016,), jnp.int32),
        pltpu.SemaphoreType.DMA,
    ],
)
def _sc_gather_deg(h_hbm, dst_hbm, nt_hbm, x0_hbm, pdeg_hbm,
                   hbuf, rows, cnt, dbuf, sem):
    wid = _wid()
    rows_per = NP // NW
    for q in range(4):
        base = _al(wid * rows_per + q * 392)
        pltpu.sync_copy(h_hbm.at[pl.ds(base, 392)], hbuf)
        pltpu.async_copy(nt_hbm.at[hbuf], rows, sem).wait()
        pltpu.sync_copy(rows, x0_hbm.at[pl.ds(base, 392), :])

    def zz(c, carry):
        cnt[pl.ds(_al(c * 16), 16)] = jnp.zeros((16,), jnp.int32)
        return carry
    lax.fori_loop(0, (NP + 16) // 16, zz, 0)

    ones = jnp.ones((16,), jnp.int32)
    ebase = wid * EC

    def count_groups(ngr, nvalid_last):
        def gb(g, carry):
            idx = dbuf[pl.ds(_al(g * 16), 16)]
            idx = jnp.minimum(jnp.maximum(idx, 0), NP - 1)
            plsc.addupdate_scatter(cnt, [idx], ones)
            return carry
        lax.fori_loop(0, ngr, gb, 0)
        if nvalid_last:
            g = ngr
            idx = dbuf[pl.ds(_al(g * 16), 16)]
            idx = jnp.minimum(jnp.maximum(idx, 0), NP - 1)
            valid = lax.broadcasted_iota(jnp.int32, (16,), 0) < nvalid_last
            idx = jnp.where(valid, idx, NP)
            plsc.addupdate_scatter(cnt, [idx], ones)

    def chunk(ci, carry):
        pltpu.sync_copy(dst_hbm.at[pl.ds(_al(ebase + ci * 2000), 2000)],
                        dbuf.at[pl.ds(0, 2000)])
        count_groups(125, 0)
        return carry
    lax.fori_loop(0, 12, chunk, 0)
    pltpu.sync_copy(dst_hbm.at[pl.ds(_al(ebase + 24000), 1000)],
                    dbuf.at[pl.ds(0, 1000)])
    count_groups(62, 8)

    pltpu.sync_copy(cnt, pdeg_hbm.at[wid])


def _tc_proj_body(x0_ref, pdeg_ref, ws_ref, wd_ref, etab_ref, weall_ref,
                  preb_ref, xs_ref, xd_ref, deg_ref, norm_ref, tc_ref,
                  etall_ref):
    j = pl.program_id(0)
    pd = pdeg_ref[...].astype(jnp.float32)
    deg = jnp.sum(pd, axis=0)
    degc = jnp.maximum(deg, 1.0)
    deg_ref[0, 0, :] = deg
    norm_ref[0, 0, :] = lax.rsqrt(degc)
    tc_ref[0, 0, :] = jnp.sum(pdeg_ref[...], axis=1)
    x0 = x0_ref[...]
    xs_ref[...] = jnp.dot(x0, ws_ref[...], preferred_element_type=jnp.float32, precision=lax.Precision.HIGHEST)
    xd_ref[...] = jnp.dot(x0, wd_ref[...], preferred_element_type=jnp.float32, precision=lax.Precision.HIGHEST)

    @pl.when(j == 0)
    def _():
        et = etab_ref[...]
        for i in range(L):
            we = weall_ref[i]
            etall_ref[i] = (jnp.dot(et, we, preferred_element_type=jnp.float32, precision=lax.Precision.HIGHEST)
                            + preb_ref[i][None, :])


def _tc_proj(x0, pdeg, ws0, wd0, etab16, we_all, pre_b):
    return pl.pallas_call(
        _tc_proj_body,
        grid=(NB,),
        in_specs=[
            pl.BlockSpec((W, HID), lambda j: (j, 0)),
            pl.BlockSpec((NW, W), lambda j: (0, j)),
            pl.BlockSpec((HID, HID), lambda j: (0, 0)),
            pl.BlockSpec((HID, HID), lambda j: (0, 0)),
            pl.BlockSpec((16, 16), lambda j: (0, 0)),
            pl.BlockSpec((L, 16, HID), lambda j: (0, 0, 0)),
            pl.BlockSpec((L, HID), lambda j: (0, 0)),
        ],
        out_specs=[
            pl.BlockSpec((W, HID), lambda j: (j, 0)),
            pl.BlockSpec((W, HID), lambda j: (j, 0)),
            pl.BlockSpec((1, 1, W), lambda j: (j, 0, 0)),
            pl.BlockSpec((1, 1, W), lambda j: (j, 0, 0)),
            pl.BlockSpec((1, 1, NW), lambda j: (j, 0, 0)),
            pl.BlockSpec((L, 16, HID), lambda j: (0, 0, 0)),
        ],
        out_shape=[
            jax.ShapeDtypeStruct((NP, HID), jnp.float32),
            jax.ShapeDtypeStruct((NP, HID), jnp.float32),
            jax.ShapeDtypeStruct((NB, 1, W), jnp.float32),
            jax.ShapeDtypeStruct((NB, 1, W), jnp.float32),
            jax.ShapeDtypeStruct((NB, 1, NW), jnp.int32),
            jax.ShapeDtypeStruct((L, 16, HID), jnp.float32),
        ],
    )(x0, pdeg, ws0, wd0, etab16, we_all, pre_b)


@functools.partial(
    pl.kernel,
    out_type=(
        jax.ShapeDtypeStruct((EP3,), jnp.int32),
        jax.ShapeDtypeStruct((EP3,), jnp.float32),
    ),
    mesh=MESH,
    compiler_params=SC_PARAMS,
    scratch_types=[
        pltpu.VMEM((NP + 16,), jnp.float32),
        pltpu.VMEM((200,), jnp.int32),
        pltpu.VMEM((1024,), jnp.int32),
        pltpu.VMEM((1024,), jnp.int32),
        pltpu.VMEM((1024,), jnp.int32),
        pltpu.VMEM((1024,), jnp.int32),
        pltpu.VMEM((1024,), jnp.int32),
        pltpu.VMEM((1024,), jnp.int32),
        pltpu.VMEM((8, 128), jnp.float32),
        pltpu.VMEM((8, 128), jnp.int32),
        pltpu.VMEM((8, 128), jnp.int32),
        pltpu.SemaphoreType.DMA,
        pltpu.SemaphoreType.DMA,
    ],
)
def _sc_bucket(src_hbm, dst_hbm, e_hbm, norm_hbm, off_hbm,
               pkb_hbm, wb_hbm,
               normv, cur, svA, dvA, evA, svB, dvB, evB,
               wv, pkv, posv, sem, lsem):
    wid = _wid()
    pltpu.sync_copy(norm_hbm, normv)
    pltpu.sync_copy(off_hbm.at[wid], cur)

    iota = lax.broadcasted_iota(jnp.int32, (16,), 0)
    ones = jnp.ones((16,), jnp.int32)
    ebase = wid * EC
    NSUP = EC // 1024

    def fire_loads(u, sv, dv, ev):
        cbase = _al(ebase + u * 1024)
        pltpu.async_copy(src_hbm.at[pl.ds(cbase, 1024)], sv, lsem)
        pltpu.async_copy(dst_hbm.at[pl.ds(cbase, 1024)], dv, lsem)
        pltpu.async_copy(e_hbm.at[pl.ds(cbase, 1024)], ev, lsem)

    def wait_loads(u, sv, dv, ev):
        cbase = _al(ebase + u * 1024)
        pltpu.make_async_copy(src_hbm.at[pl.ds(cbase, 1024)], sv, lsem).wait()
        pltpu.make_async_copy(dst_hbm.at[pl.ds(cbase, 1024)], dv, lsem).wait()
        pltpu.make_async_copy(e_hbm.at[pl.ds(cbase, 1024)], ev, lsem).wait()

    def do_group(sv, dv, ev, r, g, nvalid):
        sl = pl.ds(r * 128 + g * 16, 16)
        s16 = sv[sl]
        d16 = dv[sl]
        e16 = ev[sl]
        s16c = jnp.minimum(jnp.maximum(s16, 0), NP - 1)
        d16c = jnp.minimum(jnp.maximum(d16, 0), NP - 1)
        ns = plsc.load_gather(normv, [s16c])
        nd = plsc.load_gather(normv, [d16c])
        gsl = pl.ds(g * 16, 16)
        wv[r, gsl] = ns * nd
        pkv[r, gsl] = (s16c | ((d16c & (W - 1)) << 16)
                       | ((jnp.minimum(jnp.maximum(e16, 0), 15)) << 24))
        b = d16c >> 8
        if nvalid is not None:
            b = jnp.where(iota < nvalid, b, 196)
        rank = jnp.zeros((16,), jnp.int32)
        for i in range(15):
            bi = b[i]
            hit = jnp.logical_and(b == bi, iota > i)
            rank = rank + jnp.where(hit, 1, 0)
        pos = plsc.load_gather(cur, [b]) + rank
        plsc.addupdate_scatter(cur, [b], ones)
        posv[r, gsl] = pos

    def fire_scatter(r):
        return (pltpu.async_copy(pkv.at[r], pkb_hbm.at[posv.at[r]], sem),
                pltpu.async_copy(wv.at[r], wb_hbm.at[posv.at[r]], sem))

    def process_super(sv, dv, ev):
        handles = []
        for r in range(8):
            for g in range(8):
                do_group(sv, dv, ev, r, g, None)
            handles.extend(fire_scatter(r))
        for hh in handles:
            hh.wait()

    fire_loads(0, svA, dvA, evA)

    def pair(v, carry):
        u0 = 2 * v
        wait_loads(u0, svA, dvA, evA)
        fire_loads(u0 + 1, svB, dvB, evB)
        process_super(svA, dvA, evA)
        wait_loads(u0 + 1, svB, dvB, evB)

        @pl.when(u0 + 2 < NSUP)
        def _():
            fire_loads(u0 + 2, svA, dvA, evA)
        process_super(svB, dvB, evB)
        return carry
    lax.fori_loop(0, NSUP // 2, pair, 0)

    rbase = _al(ebase + NSUP * 1024)
    pltpu.sync_copy(src_hbm.at[pl.ds(rbase, 424)], svA.at[pl.ds(0, 424)])
    pltpu.sync_copy(dst_hbm.at[pl.ds(rbase, 424)], dvA.at[pl.ds(0, 424)])
    pltpu.sync_copy(e_hbm.at[pl.ds(rbase, 424)], evA.at[pl.ds(0, 424)])
    handles = []
    for r in range(3):
        for g in range(8):
            do_group(svA, dvA, evA, r, g, None)
        handles.extend(fire_scatter(r))
    do_group(svA, dvA, evA, 3, 0, None)
    do_group(svA, dvA, evA, 3, 1, None)
    do_group(svA, dvA, evA, 3, 2, 8)
    for g in range(3, 8):
        posv[3, pl.ds(g * 16, 16)] = jnp.full((16,), EP2 + 64, jnp.int32) + iota
    handles.extend(fire_scatter(3))
    for hh in handles:
        hh.wait()


@functools.partial(
    pl.kernel,
    out_type=tuple(jax.ShapeDtypeStruct((NP, HID), jnp.float32)
                   for _ in range(4)),
    mesh=MESH,
    compiler_params=SC_PARAMS,
    scratch_types=[
        pltpu.VMEM((W + 1, HID), jnp.float32),
        pltpu.VMEM((W + 1, HID), jnp.float32),
        pltpu.VMEM((W + 1, HID), jnp.float32),
        pltpu.VMEM((W + 1, HID), jnp.float32),
        pltpu.VMEM((224,), jnp.int32),
        pltpu.VMEM((224,), jnp.int32),
        pltpu.SMEM((224,), jnp.int32),
        pltpu.SMEM((224,), jnp.int32),
        pltpu.VMEM((256,), jnp.int32),
        pltpu.SMEM((256,), jnp.int32),
        pltpu.VMEM((256,), jnp.int32),
        pltpu.VMEM((256,), jnp.int32),
        pltpu.VMEM((256,), jnp.float32),
        pltpu.VMEM((256,), jnp.float32),
        pltpu.VMEM((256,), jnp.int32),
        pltpu.VMEM((256,), jnp.int32),
        pltpu.VMEM((256,), jnp.int32),
        pltpu.VMEM((256,), jnp.int32),
        pltpu.VMEM((256,), jnp.int32),
        pltpu.VMEM((256,), jnp.int32),
        pltpu.VMEM((256, HID), jnp.float32),
        pltpu.VMEM((256, HID), jnp.float32),
        pltpu.SemaphoreType.DMA,
        pltpu.SemaphoreType.DMA,
        pltpu.SemaphoreType.DMA,
    ],
)
def _sc_edge_pass(xs_hbm, xd_hbm, et_hbm, pkb_hbm, wb_hbm,
                  starts_hbm, counts_hbm, asgn_hbm,
                  s1_hbm, s2_hbm, mx_hbm, mn_hbm,
                  a1, a2, amx, amn, stv, cntv, ssm, csm, agv, asm,
                  pk0, pk1, w0, w1, si0, si1, di0, di1, ei0, ei1,
                  rowsA, rowsB, gsemA, gsemB, msem):
    wid = _wid()
    pltpu.sync_copy(starts_hbm, stv)
    pltpu.sync_copy(counts_hbm, cntv)
    pltpu.sync_copy(asgn_hbm, agv)
    for g in range(14):
        svec = stv[pl.ds(g * 16, 16)]
        cvec = cntv[pl.ds(g * 16, 16)]
        for jj in range(16):
            ssm[g * 16 + jj] = svec[jj]
            csm[g * 16 + jj] = cvec[jj]
    for g in range(16):
        avec = agv[pl.ds(g * 16, 16)]
        for jj in range(16):
            asm[g * 16 + jj] = avec[jj]

    iota = lax.broadcasted_iota(jnp.int32, (16,), 0)
    zf = jnp.zeros((16,), jnp.float32)
    big = jnp.full((16,), BIGF, jnp.float32)

    def fire_meta(start, ci, pkr, wr):
        cbase = _al(start + ci * 256)
        pltpu.async_copy(pkb_hbm.at[pl.ds(cbase, 256)], pkr, msem)
        pltpu.async_copy(wb_hbm.at[pl.ds(cbase, 256)], wr, msem)

    def wait_meta(start, ci, pkr, wr):
        cbase = _al(start + ci * 256)
        pltpu.make_async_copy(pkb_hbm.at[pl.ds(cbase, 256)], pkr, msem).wait()
        pltpu.make_async_copy(wb_hbm.at[pl.ds(cbase, 256)], wr, msem).wait()

    def prep_gather(bW, pkr, sir, dir_, eir, rowsr, gsem):
        def up(g, c):
            sl = pl.ds(_al(g * 16), 16)
            v = pkr[sl]
            sir[sl] = jnp.minimum(v & 0xFFFF, NP - 1)
            dir_[sl] = bW + ((v >> 16) & 255)
            eir[sl] = (v >> 24) & 15
            return c
        lax.fori_loop(0, 16, up, 0)
        pltpu.async_copy(xs_hbm.at[sir], rowsr, gsem)

    def wait_xs(sir, rowsr, gsem):
        pltpu.make_async_copy(xs_hbm.at[sir], rowsr, gsem).wait()

    def fire_adds(dir_, eir, rowsr, gsem):
        pltpu.async_copy(xd_hbm.at[dir_], rowsr, gsem, add=True)
        pltpu.async_copy(et_hbm.at[eir], rowsr, gsem, add=True)

    def wait_adds(dir_, eir, rowsr, gsem):
        pltpu.make_async_copy(xd_hbm.at[dir_], rowsr, gsem).wait()
        pltpu.make_async_copy(et_hbm.at[eir], rowsr, gsem).wait()

    def compute(cnt, ci, pkr, wr, rowsr):
        csize = jnp.minimum(cnt - ci * 256, 256)
        ngroups = (csize + 15) >> 4

        def group_body(g, c5):
            gb = _al(g * 16)
            pk16 = pkr[pl.ds(gb, 16)]
            dl = (pk16 >> 16) & 255
            valid = (gb + iota) < csize
            dlF = jnp.where(valid, dl, W)
            w16 = jnp.where(valid, wr[pl.ds(gb, 16)], 0.0)
            for jj in range(16):
                d_j = dlF[jj]
                w_j = w16[jj]
                for k in range(4):
                    sl = pl.ds(k * 16, 16)
                    u = rowsr[gb + jj, sl]
                    m = jnp.maximum(u, 0.0) * w_j
                    a1[d_j, sl] = a1[d_j, sl] + m
                    a2[d_j, sl] = a2[d_j, sl] + m * m
                    amx[d_j, sl] = jnp.maximum(amx[d_j, sl], m)
                    amn[d_j, sl] = jnp.minimum(amn[d_j, sl], m)
            return c5
        lax.fori_loop(0, ngroups, group_body, 0)

    def bucket_body(t, carry):
        b = asm[wid * 8 + t]

        @pl.when(b >= 0)
        def _():
            start = ssm[b]
            cnt = csm[b]

            def zrow(r, c2):
                for k in range(4):
                    sl = pl.ds(k * 16, 16)
                    a1[r, sl] = zf
                    a2[r, sl] = zf
                    amx[r, sl] = zf
                    amn[r, sl] = big
                return c2
            lax.fori_loop(0, W + 1, zrow, 0)

            nchunks = (cnt + 255) >> 8
            bW = b * W

            @pl.when(nchunks > 0)
            def _():
                fire_meta(start, 0, pk0, w0)
                wait_meta(start, 0, pk0, w0)
                prep_gather(bW, pk0, si0, di0, ei0, rowsA, gsemA)

                @pl.when(nchunks > 1)
                def _():
                    fire_meta(start, 1, pk1, w1)
                wait_xs(si0, rowsA, gsemA)
                fire_adds(di0, ei0, rowsA, gsemA)

                def super_body(s, c3):
                    c0 = 2 * s
                    c1 = c0 + 1
                    c2 = c0 + 2
                    c3_ = c0 + 3

                    @pl.when(c1 < nchunks)
                    def _():
                        wait_meta(start, c1, pk1, w1)
                        prep_gather(bW, pk1, si1, di1, ei1, rowsB, gsemB)
                    wait_adds(di0, ei0, rowsA, gsemA)
                    compute(cnt, c0, pk0, w0, rowsA)

                    @pl.when(c2 < nchunks)
                    def _():
                        fire_meta(start, c2, pk0, w0)

                    @pl.when(c1 < nchunks)
                    def _():
                        wait_xs(si1, rowsB, gsemB)
                        fire_adds(di1, ei1, rowsB, gsemB)

                    @pl.when(c2 < nchunks)
                    def _():
                        wait_meta(start, c2, pk0, w0)
                        prep_gather(bW, pk0, si0, di0, ei0, rowsA, gsemA)

                    @pl.when(c1 < nchunks)
                    def _():
                        wait_adds(di1, ei1, rowsB, gsemB)
                        compute(cnt, c1, pk1, w1, rowsB)

                    @pl.when(c3_ < nchunks)
                    def _():
                        fire_meta(start, c3_, pk1, w1)

                    @pl.when(c2 < nchunks)
                    def _():
                        wait_xs(si0, rowsA, gsemA)
                        fire_adds(di0, ei0, rowsA, gsemA)
                    return c3
                lax.fori_loop(0, (nchunks + 1) >> 1, super_body, 0)

            row_sl = pl.ds(0, W)
            out_sl = pl.ds(b * W, W)
            pltpu.sync_copy(a1.at[row_sl, :], s1_hbm.at[out_sl, :])
            pltpu.sync_copy(a2.at[row_sl, :], s2_hbm.at[out_sl, :])
            pltpu.sync_copy(amx.at[row_sl, :], mx_hbm.at[out_sl, :])
            pltpu.sync_copy(amn.at[row_sl, :], mn_hbm.at[out_sl, :])
        return carry
    lax.fori_loop(0, 8, bucket_body, 0)


def _tc_agg_body(x_ref, s1_ref, s2_ref, mx_ref, mn_ref, deg_ref,
                 p0_ref, pa_ref, pb_ref, pc_ref, pbias_ref,
                 out_ref, stats_ref):
    j = pl.program_id(0)
    deg = deg_ref[0, 0, :][:, None]
    degc = jnp.maximum(deg, 1.0)
    invd = 1.0 / degc
    hase = deg > 0.0
    logd = jnp.log(degc + 1.0)
    amp = logd * (1.0 / AVG_D_LOG)
    att = AVG_D_LOG / logd
    mean = s1_ref[...] * invd
    std = jnp.sqrt(jnp.maximum(s2_ref[...] * invd - mean * mean, 0.0) + 1e-5)
    mx = jnp.where(hase, mx_ref[...], 0.0)
    mn = jnp.where(hase, mn_ref[...], 0.0)
    A = jnp.concatenate([mean, mx, mn, std], axis=1)
    out = (jnp.dot(x_ref[...], p0_ref[...], preferred_element_type=jnp.float32, precision=lax.Precision.HIGHEST)
           + jnp.dot(A, pa_ref[...], preferred_element_type=jnp.float32, precision=lax.Precision.HIGHEST)
           + jnp.dot(A * amp, pb_ref[...], preferred_element_type=jnp.float32, precision=lax.Precision.HIGHEST)
           + jnp.dot(A * att, pc_ref[...], preferred_element_type=jnp.float32, precision=lax.Precision.HIGHEST)
           + pbias_ref[...])
    out_ref[...] = out
    rowid = j * W + lax.broadcasted_iota(jnp.int32, (W, 1), 0)
    om = jnp.where(rowid < N, out, 0.0)

    @pl.when(j == 0)
    def _():
        stats_ref[...] = jnp.zeros_like(stats_ref)
    stats_ref[0, :] = stats_ref[0, :] + jnp.sum(om, axis=0)
    stats_ref[1, :] = stats_ref[1, :] + jnp.sum(om * om, axis=0)


def _tc_agg(x, s1, s2, mx, mn, deg3, p0, pa, pb, pc, pbias):
    blk = lambda j: (j, 0)
    return pl.pallas_call(
        _tc_agg_body,
        grid=(NB,),
        in_specs=[
            pl.BlockSpec((W, HID), blk),
            pl.BlockSpec((W, HID), blk),
            pl.BlockSpec((W, HID), blk),
            pl.BlockSpec((W, HID), blk),
            pl.BlockSpec((W, HID), blk),
            pl.BlockSpec((1, 1, W), lambda j: (j, 0, 0)),
            pl.BlockSpec((HID, HID), lambda j: (0, 0)),
            pl.BlockSpec((4 * HID, HID), lambda j: (0, 0)),
            pl.BlockSpec((4 * HID, HID), lambda j: (0, 0)),
            pl.BlockSpec((4 * HID, HID), lambda j: (0, 0)),
            pl.BlockSpec((1, HID), lambda j: (0, 0)),
        ],
        out_specs=[
            pl.BlockSpec((W, HID), blk),
            pl.BlockSpec((2, HID), lambda j: (0, 0)),
        ],
        out_shape=[
            jax.ShapeDtypeStruct((NP, HID), jnp.float32),
            jax.ShapeDtypeStruct((2, HID), jnp.float32),
        ],
    )(x, s1, s2, mx, mn, deg3, p0, pa, pb, pc, pbias)


def _tc_bn_proj_body(x_ref, o_ref, mu_ref, inv_ref, g_ref, b_ref,
                     ws_ref, wd_ref, xn_ref, xs_ref, xd_ref):
    o = (o_ref[...] - mu_ref[...]) * inv_ref[...] * g_ref[...] + b_ref[...]
    xn = x_ref[...] + jnp.maximum(o, 0.0)
    xn_ref[...] = xn
    xs_ref[...] = jnp.dot(xn, ws_ref[...], preferred_element_type=jnp.float32, precision=lax.Precision.HIGHEST)
    xd_ref[...] = jnp.dot(xn, wd_ref[...], preferred_element_type=jnp.float32, precision=lax.Precision.HIGHEST)


def _tc_bn_proj(x, out_pre, mu, inv, gam, bet, ws, wd):
    blk = lambda j: (j, 0)
    one = lambda j: (0, 0)
    return pl.pallas_call(
        _tc_bn_proj_body,
        grid=(NB,),
        in_specs=[
            pl.BlockSpec((W, HID), blk),
            pl.BlockSpec((W, HID), blk),
            pl.BlockSpec((1, HID), one),
            pl.BlockSpec((1, HID), one),
            pl.BlockSpec((1, HID), one),
            pl.BlockSpec((1, HID), one),
            pl.BlockSpec((HID, HID), one),
            pl.BlockSpec((HID, HID), one),
        ],
        out_specs=[pl.BlockSpec((W, HID), blk)] * 3,
        out_shape=[jax.ShapeDtypeStruct((NP, HID), jnp.float32)] * 3,
    )(x, out_pre, mu, inv, gam, bet, ws, wd)


def _tc_bn_pool_body(x_ref, o_ref, mu_ref, inv_ref, g_ref, b_ref, hsum_ref):
    j = pl.program_id(0)
    o = (o_ref[...] - mu_ref[...]) * inv_ref[...] * g_ref[...] + b_ref[...]
    xn = x_ref[...] + jnp.maximum(o, 0.0)
    rowid = j * W + lax.broadcasted_iota(jnp.int32, (W, 1), 0)
    xm = jnp.where(rowid < N, xn, 0.0)

    @pl.when(j == 0)
    def _():
        hsum_ref[...] = jnp.zeros_like(hsum_ref)
    hsum_ref[0, :] = hsum_ref[0, :] + jnp.sum(xm, axis=0)


def _tc_bn_pool(x, out_pre, mu, inv, gam, bet):
    blk = lambda j: (j, 0)
    one = lambda j: (0, 0)
    return pl.pallas_call(
        _tc_bn_pool_body,
        grid=(NB,),
        in_specs=[
            pl.BlockSpec((W, HID), blk),
            pl.BlockSpec((W, HID), blk),
            pl.BlockSpec((1, HID), one),
            pl.BlockSpec((1, HID), one),
            pl.BlockSpec((1, HID), one),
            pl.BlockSpec((1, HID), one),
        ],
        out_specs=pl.BlockSpec((1, HID), one),
        out_shape=jax.ShapeDtypeStruct((1, HID), jnp.float32),
    )(x, out_pre, mu, inv, gam, bet)


def _tc_readout_body(hsum_ref, w1_ref, b1_ref, w2_ref, b2_ref, w3_ref, b3_ref,
                     o_ref):
    hg = hsum_ref[...] * (1.0 / N)
    z = jnp.maximum(jnp.dot(hg, w1_ref[...],
                            preferred_element_type=jnp.float32, precision=lax.Precision.HIGHEST) + b1_ref[...],
                    0.0)
    z = jnp.maximum(jnp.dot(z, w2_ref[...],
                            preferred_element_type=jnp.float32, precision=lax.Precision.HIGHEST) + b2_ref[...],
                    0.0)
    o_ref[...] = jnp.dot(z, w3_ref[...],
                         preferred_element_type=jnp.float32, precision=lax.Precision.HIGHEST) + b3_ref[...]


def _tc_readout(hsum, rW1, rb1, rW2, rb2, rW3, rb3):
    return pl.pallas_call(
        _tc_readout_body,
        out_shape=jax.ShapeDtypeStruct((1, 1), jnp.float32),
    )(hsum, rW1, rb1[None, :], rW2, rb2[None, :], rW3, rb3[None, :])


def kernel(h, e, edge_index, node_table, edge_table, pre_W, pre_b, post_W,
           post_b, gamma, beta, rW1, rb1, rW2, rb2, rW3, rb3):
    src = edge_index[0].astype(jnp.int32)
    dst = edge_index[1].astype(jnp.int32)
    e32 = e.astype(jnp.int32)
    h_pad = jnp.pad(h.astype(jnp.int32), (0, NP - N))
    etab16 = jnp.pad(edge_table, ((0, 6), (0, 0)))

    we_all = pre_W[:, 128:144, :]

    x0, pdeg_raw = _sc_gather_deg(h_pad, dst, node_table)
    pdeg = pdeg_raw[:, :NP]

    xs, xd, deg3, norm3, tc3, et_all = _tc_proj(
        x0, pdeg, pre_W[0, :64, :], pre_W[0, 64:128, :], etab16, we_all, pre_b)

    counts = jnp.sum(tc3[:, 0, :], axis=1)
    padded = ((counts + 7) // 8) * 8
    starts = jnp.concatenate([jnp.zeros((1,), jnp.int32),
                              jnp.cumsum(padded)[:-1].astype(jnp.int32)])
    tilecnt = tc3[:, 0, :].T
    excl = jnp.cumsum(tilecnt, axis=0) - tilecnt
    offsets = starts[None, :] + excl.astype(jnp.int32)
    offsets = jnp.concatenate(
        [offsets, jnp.full((NW, 4), EP2, jnp.int32)], axis=1)
    starts_pad = jnp.pad(starts, (0, 224 - NB))
    counts_pad = jnp.pad(counts.astype(jnp.int32), (0, 224 - NB))
    norm_flat = jnp.pad(norm3.reshape(NP), (0, 16))

    order = jnp.argsort(-counts).astype(jnp.int32)
    ii = jnp.arange(NB, dtype=jnp.int32)
    row = ii // NW
    col = ii % NW
    tile = jnp.where(row % 2 == 0, col, NW - 1 - col)
    asgn = jnp.full((NW, 8), -1, jnp.int32).at[tile, row].set(order)
    asgn_flat = asgn.reshape(NW * 8)

    pk_b, w_b = _sc_bucket(src, dst, e32, norm_flat, offsets)

    x = x0
    for i in range(L):
        s1, s2, mx, mn = _sc_edge_pass(
            xs, xd, et_all[i], pk_b, w_b, starts_pad, counts_pad, asgn_flat)
        P = post_W[i]
        blkP = lambda k: P[64 * k:64 * (k + 1)]
        p0 = blkP(0)
        pa = jnp.concatenate([blkP(1), blkP(4), blkP(7), blkP(10)], axis=0)
        pb = jnp.concatenate([blkP(2), blkP(5), blkP(8), blkP(11)], axis=0)
        pc = jnp.concatenate([blkP(3), blkP(6), blkP(9), blkP(12)], axis=0)
        out_pre, stats = _tc_agg(x, s1, s2, mx, mn, deg3, p0, pa, pb, pc,
                                 post_b[i][None, :])
        mu = stats[0] * (1.0 / N)
        var = stats[1] * (1.0 / N) - mu * mu
        inv = lax.rsqrt(var + 1e-5)
        if i < L - 1:
            x, xs, xd = _tc_bn_proj(x, out_pre, mu[None, :], inv[None, :],
                                    gamma[i][None, :], beta[i][None, :],
                                    pre_W[i + 1, :64, :],
                                    pre_W[i + 1, 64:128, :])
        else:
            hsum = _tc_bn_pool(x, out_pre, mu[None, :], inv[None, :],
                               gamma[i][None, :], beta[i][None, :])
    return _tc_readout(hsum, rW1, rb1, rW2, rb2, rW3, rb3)


# speedup vs baseline: 1.9771x; 1.9771x over previous
import functools

import jax
import jax.numpy as jnp
import numpy as np
from jax import lax
from jax.experimental import pallas as pl
from jax.experimental.pallas import tpu as pltpu
from jax.experimental.pallas import tpu_sc as plsc

N = 50000
E = 800000
HID = 64
L = 4
AVG_D_LOG = float(np.log(16.0))

W = 256
NB = 196
NP = NB * W
NW = 32
EC = E // NW
EP2 = E + NB * 8
EP3 = EP2 + 512
BIGF = 3.0e38

MESH = plsc.VectorSubcoreMesh(core_axis_name="c", subcore_axis_name="s")
SC_PARAMS = pltpu.CompilerParams(use_tc_tiling_on_sc=False,
                                 needs_layout_passes=False)

_IOTA16 = tuple(range(16))


def _al(x):
    return pl.multiple_of(x, 8)


def _wid():
    return lax.axis_index("s") * 2 + lax.axis_index("c")


@functools.partial(
    pl.kernel,
    out_type=(
        jax.ShapeDtypeStruct((NP, HID), jnp.float32),
        jax.ShapeDtypeStruct((NW, NP + 16), jnp.int32),
    ),
    mesh=MESH,
    compiler_params=SC_PARAMS,
    scratch_types=[
        pltpu.VMEM((392,), jnp.int32),
        pltpu.VMEM((392, HID), jnp.float32),
        pltpu.VMEM((NP + 16,), jnp.int32),
        pltpu.VMEM((2016,), jnp.int32),
        pltpu.SemaphoreType.DMA,
    ],
)
def _sc_gather_deg(h_hbm, dst_hbm, nt_hbm, x0_hbm, pdeg_hbm,
                   hbuf, rows, cnt, dbuf, sem):
    wid = _wid()
    rows_per = NP // NW
    for q in range(4):
        base = _al(wid * rows_per + q * 392)
        pltpu.sync_copy(h_hbm.at[pl.ds(base, 392)], hbuf)
        pltpu.async_copy(nt_hbm.at[hbuf], rows, sem).wait()
        pltpu.sync_copy(rows, x0_hbm.at[pl.ds(base, 392), :])

    def zz(c, carry):
        cnt[pl.ds(_al(c * 16), 16)] = jnp.zeros((16,), jnp.int32)
        return carry
    lax.fori_loop(0, (NP + 16) // 16, zz, 0)

    ones = jnp.ones((16,), jnp.int32)
    ebase = wid * EC

    def count_groups(ngr, nvalid_last):
        def gb(g, carry):
            idx = dbuf[pl.ds(_al(g * 16), 16)]
            idx = jnp.minimum(jnp.maximum(idx, 0), NP - 1)
            plsc.addupdate_scatter(cnt, [idx], ones)
            return carry
        lax.fori_loop(0, ngr, gb, 0)
        if nvalid_last:
            g = ngr
            idx = dbuf[pl.ds(_al(g * 16), 16)]
            idx = jnp.minimum(jnp.maximum(idx, 0), NP - 1)
            valid = lax.broadcasted_iota(jnp.int32, (16,), 0) < nvalid_last
            idx = jnp.where(valid, idx, NP)
            plsc.addupdate_scatter(cnt, [idx], ones)

    def chunk(ci, carry):
        pltpu.sync_copy(dst_hbm.at[pl.ds(_al(ebase + ci * 2000), 2000)],
                        dbuf.at[pl.ds(0, 2000)])
        count_groups(125, 0)
        return carry
    lax.fori_loop(0, 12, chunk, 0)
    pltpu.sync_copy(dst_hbm.at[pl.ds(_al(ebase + 24000), 1000)],
                    dbuf.at[pl.ds(0, 1000)])
    count_groups(62, 8)

    pltpu.sync_copy(cnt, pdeg_hbm.at[wid])


def _tc_proj_body(x0_ref, pdeg_ref, ws_ref, wd_ref, etab_ref, weall_ref,
                  preb_ref, xs_ref, xd_ref, deg_ref, norm_ref, tc_ref,
                  etall_ref):
    j = pl.program_id(0)
    pd = pdeg_ref[...].astype(jnp.float32)
    deg = jnp.sum(pd, axis=0)
    degc = jnp.maximum(deg, 1.0)
    deg_ref[0, 0, :] = deg
    norm_ref[0, 0, :] = lax.rsqrt(degc)
    tc_ref[0, 0, :] = jnp.sum(pdeg_ref[...], axis=1)
    x0 = x0_ref[...]
    xs_ref[...] = jnp.dot(x0, ws_ref[...], preferred_element_type=jnp.float32, precision=lax.Precision.HIGHEST)
    xd_ref[...] = jnp.dot(x0, wd_ref[...], preferred_element_type=jnp.float32, precision=lax.Precision.HIGHEST)

    @pl.when(j == 0)
    def _():
        et = etab_ref[...]
        for i in range(L):
            we = weall_ref[i]
            etall_ref[i] = (jnp.dot(et, we, preferred_element_type=jnp.float32, precision=lax.Precision.HIGHEST)
                            + preb_ref[i][None, :])


def _tc_proj(x0, pdeg, ws0, wd0, etab16, we_all, pre_b):
    return pl.pallas_call(
        _tc_proj_body,
        grid=(NB,),
        in_specs=[
            pl.BlockSpec((W, HID), lambda j: (j, 0)),
            pl.BlockSpec((NW, W), lambda j: (0, j)),
            pl.BlockSpec((HID, HID), lambda j: (0, 0)),
            pl.BlockSpec((HID, HID), lambda j: (0, 0)),
            pl.BlockSpec((16, 16), lambda j: (0, 0)),
            pl.BlockSpec((L, 16, HID), lambda j: (0, 0, 0)),
            pl.BlockSpec((L, HID), lambda j: (0, 0)),
        ],
        out_specs=[
            pl.BlockSpec((W, HID), lambda j: (j, 0)),
            pl.BlockSpec((W, HID), lambda j: (j, 0)),
            pl.BlockSpec((1, 1, W), lambda j: (j, 0, 0)),
            pl.BlockSpec((1, 1, W), lambda j: (j, 0, 0)),
            pl.BlockSpec((1, 1, NW), lambda j: (j, 0, 0)),
            pl.BlockSpec((L, 16, HID), lambda j: (0, 0, 0)),
        ],
        out_shape=[
            jax.ShapeDtypeStruct((NP, HID), jnp.float32),
            jax.ShapeDtypeStruct((NP, HID), jnp.float32),
            jax.ShapeDtypeStruct((NB, 1, W), jnp.float32),
            jax.ShapeDtypeStruct((NB, 1, W), jnp.float32),
            jax.ShapeDtypeStruct((NB, 1, NW), jnp.int32),
            jax.ShapeDtypeStruct((L, 16, HID), jnp.float32),
        ],
    )(x0, pdeg, ws0, wd0, etab16, we_all, pre_b)


@functools.partial(
    pl.kernel,
    out_type=(
        jax.ShapeDtypeStruct((EP3,), jnp.int32),
        jax.ShapeDtypeStruct((EP3,), jnp.float32),
    ),
    mesh=MESH,
    compiler_params=SC_PARAMS,
    scratch_types=[
        pltpu.VMEM((NP + 16,), jnp.float32),
        pltpu.VMEM((200,), jnp.int32),
        pltpu.VMEM((1024,), jnp.int32),
        pltpu.VMEM((1024,), jnp.int32),
        pltpu.VMEM((1024,), jnp.int32),
        pltpu.VMEM((1024,), jnp.int32),
        pltpu.VMEM((1024,), jnp.int32),
        pltpu.VMEM((1024,), jnp.int32),
        pltpu.VMEM((8, 128), jnp.float32),
        pltpu.VMEM((8, 128), jnp.int32),
        pltpu.VMEM((8, 128), jnp.int32),
        pltpu.SemaphoreType.DMA,
        pltpu.SemaphoreType.DMA,
    ],
)
def _sc_bucket(src_hbm, dst_hbm, e_hbm, norm_hbm, off_hbm,
               pkb_hbm, wb_hbm,
               normv, cur, svA, dvA, evA, svB, dvB, evB,
               wv, pkv, posv, sem, lsem):
    wid = _wid()
    pltpu.sync_copy(norm_hbm, normv)
    pltpu.sync_copy(off_hbm.at[wid], cur)

    iota = lax.broadcasted_iota(jnp.int32, (16,), 0)
    ones = jnp.ones((16,), jnp.int32)
    ebase = wid * EC
    NSUP = EC // 1024

    def fire_loads(u, sv, dv, ev):
        cbase = _al(ebase + u * 1024)
        pltpu.async_copy(src_hbm.at[pl.ds(cbase, 1024)], sv, lsem)
        pltpu.async_copy(dst_hbm.at[pl.ds(cbase, 1024)], dv, lsem)
        pltpu.async_copy(e_hbm.at[pl.ds(cbase, 1024)], ev, lsem)

    def wait_loads(u, sv, dv, ev):
        cbase = _al(ebase + u * 1024)
        pltpu.make_async_copy(src_hbm.at[pl.ds(cbase, 1024)], sv, lsem).wait()
        pltpu.make_async_copy(dst_hbm.at[pl.ds(cbase, 1024)], dv, lsem).wait()
        pltpu.make_async_copy(e_hbm.at[pl.ds(cbase, 1024)], ev, lsem).wait()

    def do_group(sv, dv, ev, r, g, nvalid):
        sl = pl.ds(r * 128 + g * 16, 16)
        s16 = sv[sl]
        d16 = dv[sl]
        e16 = ev[sl]
        s16c = jnp.minimum(jnp.maximum(s16, 0), NP - 1)
        d16c = jnp.minimum(jnp.maximum(d16, 0), NP - 1)
        ns = plsc.load_gather(normv, [s16c])
        nd = plsc.load_gather(normv, [d16c])
        gsl = pl.ds(g * 16, 16)
        wv[r, gsl] = ns * nd
        pkv[r, gsl] = (s16c | ((d16c & (W - 1)) << 16)
                       | ((jnp.minimum(jnp.maximum(e16, 0), 15)) << 24))
        b = d16c >> 8
        if nvalid is not None:
            b = jnp.where(iota < nvalid, b, 196)
        rank = jnp.zeros((16,), jnp.int32)
        for i in range(15):
            bi = b[i]
            hit = jnp.logical_and(b == bi, iota > i)
            rank = rank + jnp.where(hit, 1, 0)
        pos = plsc.load_gather(cur, [b]) + rank
        plsc.addupdate_scatter(cur, [b], ones)
        posv[r, gsl] = pos

    def fire_scatter(r):
        return (pltpu.async_copy(pkv.at[r], pkb_hbm.at[posv.at[r]], sem),
                pltpu.async_copy(wv.at[r], wb_hbm.at[posv.at[r]], sem))

    def process_super(sv, dv, ev):
        handles = []
        for r in range(8):
            for g in range(8):
                do_group(sv, dv, ev, r, g, None)
            handles.extend(fire_scatter(r))
        for hh in handles:
            hh.wait()

    fire_loads(0, svA, dvA, evA)

    def pair(v, carry):
        u0 = 2 * v
        wait_loads(u0, svA, dvA, evA)
        fire_loads(u0 + 1, svB, dvB, evB)
        process_super(svA, dvA, evA)
        wait_loads(u0 + 1, svB, dvB, evB)

        @pl.when(u0 + 2 < NSUP)
        def _():
            fire_loads(u0 + 2, svA, dvA, evA)
        process_super(svB, dvB, evB)
        return carry
    lax.fori_loop(0, NSUP // 2, pair, 0)

    rbase = _al(ebase + NSUP * 1024)
    pltpu.sync_copy(src_hbm.at[pl.ds(rbase, 424)], svA.at[pl.ds(0, 424)])
    pltpu.sync_copy(dst_hbm.at[pl.ds(rbase, 424)], dvA.at[pl.ds(0, 424)])
    pltpu.sync_copy(e_hbm.at[pl.ds(rbase, 424)], evA.at[pl.ds(0, 424)])
    handles = []
    for r in range(3):
        for g in range(8):
            do_group(svA, dvA, evA, r, g, None)
        handles.extend(fire_scatter(r))
    do_group(svA, dvA, evA, 3, 0, None)
    do_group(svA, dvA, evA, 3, 1, None)
    do_group(svA, dvA, evA, 3, 2, 8)
    for g in range(3, 8):
        posv[3, pl.ds(g * 16, 16)] = jnp.full((16,), EP2 + 64, jnp.int32) + iota
    handles.extend(fire_scatter(3))
    for hh in handles:
        hh.wait()


@functools.partial(
    pl.kernel,
    out_type=tuple(jax.ShapeDtypeStruct((NP, HID), jnp.float32)
                   for _ in range(4)),
    mesh=MESH,
    compiler_params=SC_PARAMS,
    scratch_types=[
        pltpu.VMEM((W + 1, HID), jnp.float32),
        pltpu.VMEM((W + 1, HID), jnp.float32),
        pltpu.VMEM((W + 1, HID), jnp.float32),
        pltpu.VMEM((W + 1, HID), jnp.float32),
        pltpu.VMEM((W + 1, HID), jnp.float32),
        pltpu.VMEM((16, HID), jnp.float32),
        pltpu.VMEM((224,), jnp.int32),
        pltpu.VMEM((224,), jnp.int32),
        pltpu.SMEM((224,), jnp.int32),
        pltpu.SMEM((224,), jnp.int32),
        pltpu.VMEM((256,), jnp.int32),
        pltpu.SMEM((256,), jnp.int32),
        pltpu.VMEM((256,), jnp.int32),
        pltpu.VMEM((256,), jnp.int32),
        pltpu.VMEM((256,), jnp.float32),
        pltpu.VMEM((256,), jnp.float32),
        pltpu.VMEM((256,), jnp.int32),
        pltpu.VMEM((256,), jnp.int32),
        pltpu.VMEM((256, HID), jnp.float32),
        pltpu.VMEM((256, HID), jnp.float32),
        pltpu.SemaphoreType.DMA,
        pltpu.SemaphoreType.DMA,
        pltpu.SemaphoreType.DMA,
    ],
)
def _sc_edge_pass(xs_hbm, xd_hbm, et_hbm, pkb_hbm, wb_hbm,
                  starts_hbm, counts_hbm, asgn_hbm,
                  s1_hbm, s2_hbm, mx_hbm, mn_hbm,
                  a1, a2, amx, amn, xdr, etv, stv, cntv, ssm, csm, agv, asm,
                  pk0, pk1, w0, w1, si0, si1,
                  rowsA, rowsB, gsemA, gsemB, msem):
    wid = _wid()
    pltpu.sync_copy(et_hbm, etv)
    pltpu.sync_copy(starts_hbm, stv)
    pltpu.sync_copy(counts_hbm, cntv)
    pltpu.sync_copy(asgn_hbm, agv)
    for g in range(14):
        svec = stv[pl.ds(g * 16, 16)]
        cvec = cntv[pl.ds(g * 16, 16)]
        for jj in range(16):
            ssm[g * 16 + jj] = svec[jj]
            csm[g * 16 + jj] = cvec[jj]
    for g in range(16):
        avec = agv[pl.ds(g * 16, 16)]
        for jj in range(16):
            asm[g * 16 + jj] = avec[jj]

    iota = lax.broadcasted_iota(jnp.int32, (16,), 0)
    zf = jnp.zeros((16,), jnp.float32)
    big = jnp.full((16,), BIGF, jnp.float32)

    def fire_meta(start, ci, pkr, wr):
        cbase = _al(start + ci * 256)
        pltpu.async_copy(pkb_hbm.at[pl.ds(cbase, 256)], pkr, msem)
        pltpu.async_copy(wb_hbm.at[pl.ds(cbase, 256)], wr, msem)

    def wait_meta(start, ci, pkr, wr):
        cbase = _al(start + ci * 256)
        pltpu.make_async_copy(pkb_hbm.at[pl.ds(cbase, 256)], pkr, msem).wait()
        pltpu.make_async_copy(wb_hbm.at[pl.ds(cbase, 256)], wr, msem).wait()

    def prep_gather(pkr, sir, rowsr, gsem):
        def up(g, c):
            sl = pl.ds(_al(g * 16), 16)
            v = pkr[sl] & 0xFFFF
            sir[sl] = jnp.minimum(v, NP - 1)
            return c
        lax.fori_loop(0, 16, up, 0)
        pltpu.async_copy(xs_hbm.at[sir], rowsr, gsem)

    def wait_gather(sir, rowsr, gsem):
        pltpu.make_async_copy(xs_hbm.at[sir], rowsr, gsem).wait()

    def compute(cnt, ci, pkr, wr, rowsr):
        csize = jnp.minimum(cnt - ci * 256, 256)
        ngroups = (csize + 15) >> 4

        def group_body(g, c5):
            gb = _al(g * 16)
            pk16 = pkr[pl.ds(gb, 16)]
            dl = (pk16 >> 16) & 255
            valid = (gb + iota) < csize
            dlF = jnp.where(valid, dl, W)
            e16 = (pk16 >> 24) & 15
            w16 = jnp.where(valid, wr[pl.ds(gb, 16)], 0.0)
            for jj in range(16):
                d_j = dlF[jj]
                e_j = e16[jj]
                w_j = w16[jj]
                for k in range(4):
                    sl = pl.ds(k * 16, 16)
                    u = (rowsr[gb + jj, sl] + xdr[d_j, sl]
                         + etv[e_j, sl])
                    m = jnp.maximum(u, 0.0) * w_j
                    a1[d_j, sl] = a1[d_j, sl] + m
                    a2[d_j, sl] = a2[d_j, sl] + m * m
                    amx[d_j, sl] = jnp.maximum(amx[d_j, sl], m)
                    amn[d_j, sl] = jnp.minimum(amn[d_j, sl], m)
            return c5
        lax.fori_loop(0, ngroups, group_body, 0)

    def bucket_body(t, carry):
        b = asm[wid * 8 + t]

        @pl.when(b >= 0)
        def _():
            start = ssm[b]
            cnt = csm[b]

            def zrow(r, c2):
                for k in range(4):
                    sl = pl.ds(k * 16, 16)
                    a1[r, sl] = zf
                    a2[r, sl] = zf
                    amx[r, sl] = zf
                    amn[r, sl] = big
                return c2
            lax.fori_loop(0, W + 1, zrow, 0)

            pltpu.sync_copy(xd_hbm.at[pl.ds(b * W, W), :],
                            xdr.at[pl.ds(0, W), :])

            nchunks = (cnt + 255) >> 8

            @pl.when(nchunks > 0)
            def _():
                fire_meta(start, 0, pk0, w0)
                wait_meta(start, 0, pk0, w0)
                prep_gather(pk0, si0, rowsA, gsemA)

                @pl.when(nchunks > 1)
                def _():
                    fire_meta(start, 1, pk1, w1)

                def super_body(s, c3):
                    c0 = 2 * s
                    c1 = c0 + 1
                    c2 = c0 + 2
                    c3_ = c0 + 3

                    @pl.when(c1 < nchunks)
                    def _():
                        wait_meta(start, c1, pk1, w1)
                        prep_gather(pk1, si1, rowsB, gsemB)
                    wait_gather(si0, rowsA, gsemA)
                    compute(cnt, c0, pk0, w0, rowsA)

                    @pl.when(c2 < nchunks)
                    def _():
                        fire_meta(start, c2, pk0, w0)

                    @pl.when(c1 < nchunks)
                    def _():
                        wait_gather(si1, rowsB, gsemB)
                        compute(cnt, c1, pk1, w1, rowsB)

                    @pl.when(c2 < nchunks)
                    def _():
                        wait_meta(start, c2, pk0, w0)
                        prep_gather(pk0, si0, rowsA, gsemA)

                    @pl.when(c3_ < nchunks)
                    def _():
                        fire_meta(start, c3_, pk1, w1)
                    return c3
                lax.fori_loop(0, (nchunks + 1) >> 1, super_body, 0)

            row_sl = pl.ds(0, W)
            out_sl = pl.ds(b * W, W)
            pltpu.sync_copy(a1.at[row_sl, :], s1_hbm.at[out_sl, :])
            pltpu.sync_copy(a2.at[row_sl, :], s2_hbm.at[out_sl, :])
            pltpu.sync_copy(amx.at[row_sl, :], mx_hbm.at[out_sl, :])
            pltpu.sync_copy(amn.at[row_sl, :], mn_hbm.at[out_sl, :])
        return carry
    lax.fori_loop(0, 8, bucket_body, 0)


def _tc_agg_body(x_ref, s1_ref, s2_ref, mx_ref, mn_ref, deg_ref,
                 p0_ref, pa_ref, pb_ref, pc_ref, pbias_ref,
                 out_ref, stats_ref):
    j = pl.program_id(0)
    deg = deg_ref[0, 0, :][:, None]
    degc = jnp.maximum(deg, 1.0)
    invd = 1.0 / degc
    hase = deg > 0.0
    logd = jnp.log(degc + 1.0)
    amp = logd * (1.0 / AVG_D_LOG)
    att = AVG_D_LOG / logd
    mean = s1_ref[...] * invd
    std = jnp.sqrt(jnp.maximum(s2_ref[...] * invd - mean * mean, 0.0) + 1e-5)
    mx = jnp.where(hase, mx_ref[...], 0.0)
    mn = jnp.where(hase, mn_ref[...], 0.0)
    A = jnp.concatenate([mean, mx, mn, std], axis=1)
    out = (jnp.dot(x_ref[...], p0_ref[...], preferred_element_type=jnp.float32, precision=lax.Precision.HIGHEST)
           + jnp.dot(A, pa_ref[...], preferred_element_type=jnp.float32, precision=lax.Precision.HIGHEST)
           + jnp.dot(A * amp, pb_ref[...], preferred_element_type=jnp.float32, precision=lax.Precision.HIGHEST)
           + jnp.dot(A * att, pc_ref[...], preferred_element_type=jnp.float32, precision=lax.Precision.HIGHEST)
           + pbias_ref[...])
    out_ref[...] = out
    rowid = j * W + lax.broadcasted_iota(jnp.int32, (W, 1), 0)
    om = jnp.where(rowid < N, out, 0.0)

    @pl.when(j == 0)
    def _():
        stats_ref[...] = jnp.zeros_like(stats_ref)
    stats_ref[0, :] = stats_ref[0, :] + jnp.sum(om, axis=0)
    stats_ref[1, :] = stats_ref[1, :] + jnp.sum(om * om, axis=0)


def _tc_agg(x, s1, s2, mx, mn, deg3, p0, pa, pb, pc, pbias):
    blk = lambda j: (j, 0)
    return pl.pallas_call(
        _tc_agg_body,
        grid=(NB,),
        in_specs=[
            pl.BlockSpec((W, HID), blk),
            pl.BlockSpec((W, HID), blk),
            pl.BlockSpec((W, HID), blk),
            pl.BlockSpec((W, HID), blk),
            pl.BlockSpec((W, HID), blk),
            pl.BlockSpec((1, 1, W), lambda j: (j, 0, 0)),
            pl.BlockSpec((HID, HID), lambda j: (0, 0)),
            pl.BlockSpec((4 * HID, HID), lambda j: (0, 0)),
            pl.BlockSpec((4 * HID, HID), lambda j: (0, 0)),
            pl.BlockSpec((4 * HID, HID), lambda j: (0, 0)),
            pl.BlockSpec((1, HID), lambda j: (0, 0)),
        ],
        out_specs=[
            pl.BlockSpec((W, HID), blk),
            pl.BlockSpec((2, HID), lambda j: (0, 0)),
        ],
        out_shape=[
            jax.ShapeDtypeStruct((NP, HID), jnp.float32),
            jax.ShapeDtypeStruct((2, HID), jnp.float32),
        ],
    )(x, s1, s2, mx, mn, deg3, p0, pa, pb, pc, pbias)


def _tc_bn_proj_body(x_ref, o_ref, mu_ref, inv_ref, g_ref, b_ref,
                     ws_ref, wd_ref, xn_ref, xs_ref, xd_ref):
    o = (o_ref[...] - mu_ref[...]) * inv_ref[...] * g_ref[...] + b_ref[...]
    xn = x_ref[...] + jnp.maximum(o, 0.0)
    xn_ref[...] = xn
    xs_ref[...] = jnp.dot(xn, ws_ref[...], preferred_element_type=jnp.float32, precision=lax.Precision.HIGHEST)
    xd_ref[...] = jnp.dot(xn, wd_ref[...], preferred_element_type=jnp.float32, precision=lax.Precision.HIGHEST)


def _tc_bn_proj(x, out_pre, mu, inv, gam, bet, ws, wd):
    blk = lambda j: (j, 0)
    one = lambda j: (0, 0)
    return pl.pallas_call(
        _tc_bn_proj_body,
        grid=(NB,),
        in_specs=[
            pl.BlockSpec((W, HID), blk),
            pl.BlockSpec((W, HID), blk),
            pl.BlockSpec((1, HID), one),
            pl.BlockSpec((1, HID), one),
            pl.BlockSpec((1, HID), one),
            pl.BlockSpec((1, HID), one),
            pl.BlockSpec((HID, HID), one),
            pl.BlockSpec((HID, HID), one),
        ],
        out_specs=[pl.BlockSpec((W, HID), blk)] * 3,
        out_shape=[jax.ShapeDtypeStruct((NP, HID), jnp.float32)] * 3,
    )(x, out_pre, mu, inv, gam, bet, ws, wd)


def _tc_bn_pool_body(x_ref, o_ref, mu_ref, inv_ref, g_ref, b_ref, hsum_ref):
    j = pl.program_id(0)
    o = (o_ref[...] - mu_ref[...]) * inv_ref[...] * g_ref[...] + b_ref[...]
    xn = x_ref[...] + jnp.maximum(o, 0.0)
    rowid = j * W + lax.broadcasted_iota(jnp.int32, (W, 1), 0)
    xm = jnp.where(rowid < N, xn, 0.0)

    @pl.when(j == 0)
    def _():
        hsum_ref[...] = jnp.zeros_like(hsum_ref)
    hsum_ref[0, :] = hsum_ref[0, :] + jnp.sum(xm, axis=0)


def _tc_bn_pool(x, out_pre, mu, inv, gam, bet):
    blk = lambda j: (j, 0)
    one = lambda j: (0, 0)
    return pl.pallas_call(
        _tc_bn_pool_body,
        grid=(NB,),
        in_specs=[
            pl.BlockSpec((W, HID), blk),
            pl.BlockSpec((W, HID), blk),
            pl.BlockSpec((1, HID), one),
            pl.BlockSpec((1, HID), one),
            pl.BlockSpec((1, HID), one),
            pl.BlockSpec((1, HID), one),
        ],
        out_specs=pl.BlockSpec((1, HID), one),
        out_shape=jax.ShapeDtypeStruct((1, HID), jnp.float32),
    )(x, out_pre, mu, inv, gam, bet)


def _tc_readout_body(hsum_ref, w1_ref, b1_ref, w2_ref, b2_ref, w3_ref, b3_ref,
                     o_ref):
    hg = hsum_ref[...] * (1.0 / N)
    z = jnp.maximum(jnp.dot(hg, w1_ref[...],
                            preferred_element_type=jnp.float32, precision=lax.Precision.HIGHEST) + b1_ref[...],
                    0.0)
    z = jnp.maximum(jnp.dot(z, w2_ref[...],
                            preferred_element_type=jnp.float32, precision=lax.Precision.HIGHEST) + b2_ref[...],
                    0.0)
    o_ref[...] = jnp.dot(z, w3_ref[...],
                         preferred_element_type=jnp.float32, precision=lax.Precision.HIGHEST) + b3_ref[...]


def _tc_readout(hsum, rW1, rb1, rW2, rb2, rW3, rb3):
    return pl.pallas_call(
        _tc_readout_body,
        out_shape=jax.ShapeDtypeStruct((1, 1), jnp.float32),
    )(hsum, rW1, rb1[None, :], rW2, rb2[None, :], rW3, rb3[None, :])


def kernel(h, e, edge_index, node_table, edge_table, pre_W, pre_b, post_W,
           post_b, gamma, beta, rW1, rb1, rW2, rb2, rW3, rb3):
    src = edge_index[0].astype(jnp.int32)
    dst = edge_index[1].astype(jnp.int32)
    e32 = e.astype(jnp.int32)
    h_pad = jnp.pad(h.astype(jnp.int32), (0, NP - N))
    etab16 = jnp.pad(edge_table, ((0, 6), (0, 0)))

    we_all = pre_W[:, 128:144, :]

    x0, pdeg_raw = _sc_gather_deg(h_pad, dst, node_table)
    pdeg = pdeg_raw[:, :NP]

    xs, xd, deg3, norm3, tc3, et_all = _tc_proj(
        x0, pdeg, pre_W[0, :64, :], pre_W[0, 64:128, :], etab16, we_all, pre_b)

    counts = jnp.sum(tc3[:, 0, :], axis=1)
    padded = ((counts + 7) // 8) * 8
    starts = jnp.concatenate([jnp.zeros((1,), jnp.int32),
                              jnp.cumsum(padded)[:-1].astype(jnp.int32)])
    tilecnt = tc3[:, 0, :].T
    excl = jnp.cumsum(tilecnt, axis=0) - tilecnt
    offsets = starts[None, :] + excl.astype(jnp.int32)
    offsets = jnp.concatenate(
        [offsets, jnp.full((NW, 4), EP2, jnp.int32)], axis=1)
    starts_pad = jnp.pad(starts, (0, 224 - NB))
    counts_pad = jnp.pad(counts.astype(jnp.int32), (0, 224 - NB))
    norm_flat = jnp.pad(norm3.reshape(NP), (0, 16))

    order = jnp.argsort(-counts).astype(jnp.int32)
    ii = jnp.arange(NB, dtype=jnp.int32)
    row = ii // NW
    col = ii % NW
    tile = jnp.where(row % 2 == 0, col, NW - 1 - col)
    asgn = jnp.full((NW, 8), -1, jnp.int32).at[tile, row].set(order)
    asgn_flat = asgn.reshape(NW * 8)

    pk_b, w_b = _sc_bucket(src, dst, e32, norm_flat, offsets)

    x = x0
    for i in range(L):
        s1, s2, mx, mn = _sc_edge_pass(
            xs, xd, et_all[i], pk_b, w_b, starts_pad, counts_pad, asgn_flat)
        P = post_W[i]
        blkP = lambda k: P[64 * k:64 * (k + 1)]
        p0 = blkP(0)
        pa = jnp.concatenate([blkP(1), blkP(4), blkP(7), blkP(10)], axis=0)
        pb = jnp.concatenate([blkP(2), blkP(5), blkP(8), blkP(11)], axis=0)
        pc = jnp.concatenate([blkP(3), blkP(6), blkP(9), blkP(12)], axis=0)
        out_pre, stats = _tc_agg(x, s1, s2, mx, mn, deg3, p0, pa, pb, pc,
                                 post_b[i][None, :])
        mu = stats[0] * (1.0 / N)
        var = stats[1] * (1.0 / N) - mu * mu
        inv = lax.rsqrt(var + 1e-5)
        if i < L - 1:
            x, xs, xd = _tc_bn_proj(x, out_pre, mu[None, :], inv[None, :],
                                    gamma[i][None, :], beta[i][None, :],
                                    pre_W[i + 1, :64, :],
                                    pre_W[i + 1, 64:128, :])
        else:
            hsum = _tc_bn_pool(x, out_pre, mu[None, :], inv[None, :],
                               gamma[i][None, :], beta[i][None, :])
    return _tc_readout(hsum, rW1, rb1, rW2, rb2, rW3, rb3)
